# baseline (speedup 1.0000x reference)
import functools

import jax
import jax.numpy as jnp
from jax.experimental import pallas as pl
from jax.experimental.pallas import tpu as pltpu

BT = 256
E = 8
D = 2048
I = 2048
TOPK = 2
NP = BT * TOPK
SUB = 64
NPAD = NP + E * SUB
TN = 512
NT = I // TN
TND = 512
ND = D // TND
S = NT + ND


def _routing_host(ert, rw):
    keys = ert.reshape(-1).astype(jnp.int32)
    wflat = rw.reshape(-1).astype(jnp.float32)
    eids = jnp.arange(E, dtype=jnp.int32)
    counts = jnp.sum((keys[None, :] == eids[:, None]).astype(jnp.int32), axis=1)
    rc = ((counts + SUB - 1) // SUB) * SUB
    ro = jnp.concatenate([jnp.zeros(1, jnp.int32), jnp.cumsum(rc)[:-1].astype(jnp.int32)])
    nsub = rc // SUB
    p = jnp.arange(NP, dtype=jnp.int32)
    rank = jnp.sum(((keys[:, None] == keys[None, :]) & (p[None, :] < p[:, None])).astype(jnp.int32), axis=1)
    dest = ro[keys] + rank
    perm = jnp.zeros(NPAD, jnp.int32).at[dest].set(p // TOPK)
    wsort = jnp.zeros(NPAD, jnp.float32).at[dest].set(wflat)
    fe = jax.lax.cummax(jnp.where(counts > 0, eids, 0))
    rons = jnp.stack([ro, nsub]).astype(jnp.int32)
    return perm, wsort, dest, fe.astype(jnp.int32), rons


def _moe_body(fe_ref, rons_ref, xc_ref, wu_ref, wg_ref, w2_ref, ws_ref,
              y_ref, h_ref):
    e = pl.program_id(0)
    s = pl.program_id(1)
    ro = rons_ref[0, e]
    ns = rons_ref[1, e]

    @pl.when(s < NT)
    def _():
        n = s
        wu = wu_ref[0]
        wg = wg_ref[0]

        def body(i, _):
            r0 = pl.multiple_of(ro + i * SUB, SUB)
            x = xc_ref[pl.ds(r0, SUB), :]
            u = jax.lax.dot_general(x, wu, (((1,), (1,)), ((), ())),
                                    preferred_element_type=jnp.float32)
            g = jax.lax.dot_general(x, wg, (((1,), (1,)), ((), ())),
                                    preferred_element_type=jnp.float32)
            h_ref[pl.ds(pl.multiple_of(i * SUB, SUB), SUB), pl.ds(n * TN, TN)] = (u * jax.nn.sigmoid(u)) * g
            return 0

        jax.lax.fori_loop(0, ns, body, 0)

    @pl.when(s >= NT)
    def _():
        nd = s - NT
        w2t = w2_ref[0]

        def body(i, _):
            r0 = pl.multiple_of(ro + i * SUB, SUB)
            h = h_ref[pl.ds(pl.multiple_of(i * SUB, SUB), SUB), :]
            y = jax.lax.dot_general(h, w2t, (((1,), (1,)), ((), ())),
                                    preferred_element_type=jnp.float32)
            w = ws_ref[pl.ds(r0, SUB), :]
            y_ref[pl.ds(r0, SUB), pl.ds(nd * TND, TND)] = y * w
            return 0

        jax.lax.fori_loop(0, ns, body, 0)


def kernel(hidden_states, expert_routing_table, router_weights, w13_weight, w2_weight):
    x = hidden_states.astype(jnp.float32)
    perm, wsort, dest, fe, rons = _routing_host(expert_routing_table, router_weights)
    xc = x[perm]
    ws2 = wsort[:, None]

    grid_spec = pltpu.PrefetchScalarGridSpec(
        num_scalar_prefetch=2,
        grid=(E, S),
        in_specs=[
            pl.BlockSpec((NPAD, D), lambda e, s, fe, rons: (0, 0)),
            pl.BlockSpec((1, TN, D), lambda e, s, fe, rons: (fe[e], jnp.minimum(s, NT - 1), 0)),
            pl.BlockSpec((1, TN, D), lambda e, s, fe, rons: (fe[e], NT + jnp.minimum(s, NT - 1), 0)),
            pl.BlockSpec((1, TND, D), lambda e, s, fe, rons: (fe[e], jnp.maximum(s - NT, 0), 0)),
            pl.BlockSpec((NPAD, 1), lambda e, s, fe, rons: (0, 0)),
        ],
        out_specs=pl.BlockSpec((NPAD, D), lambda e, s, fe, rons: (0, 0)),
        scratch_shapes=[pltpu.VMEM((BT, I), jnp.float32)],
    )
    ysc = pl.pallas_call(
        _moe_body,
        grid_spec=grid_spec,
        out_shape=jax.ShapeDtypeStruct((NPAD, D), jnp.float32),
    )(fe, rons, xc, w13_weight, w13_weight, w2_weight, ws2)

    pca = dest[0::2]
    pcb = dest[1::2]
    return ysc[pca] + ysc[pcb]
